# baseline (speedup 1.0000x reference)
import functools

import jax
import jax.numpy as jnp
from jax import lax
from jax.experimental import pallas as pl
from jax.experimental.pallas import tpu as pltpu
from jax.experimental.pallas import tpu_sc as plsc

N = 2048
IN_FEATURES = 128
OUT_FEATURES = 128
NUM_BANKS = 64
TOP_K = 2
TILE = 256
NTILES = N // TILE

_NUM_SC_CORES = 2
_NUM_SC_SUBCORES = 16
_NWORKERS = _NUM_SC_CORES * _NUM_SC_SUBCORES
TOK_PER_W = N // _NWORKERS


def _p_routing_sc(sel_hbm, prob_hbm, p_hbm, selv, probv, ptile):
    wid = lax.axis_index("s") * _NUM_SC_CORES + lax.axis_index("c")
    base = wid * TOK_PER_W

    zeros16 = jnp.zeros((16,), jnp.float32)
    for i in range(TOK_PER_W * NUM_BANKS // 16):
        ptile[pl.ds(i * 16, 16)] = zeros16

    lanes = lax.iota(jnp.int32, 16)
    for k in range(TOP_K):
        pltpu.sync_copy(sel_hbm.at[k, pl.ds(base, TOK_PER_W)], selv)
        pltpu.sync_copy(prob_hbm.at[k, pl.ds(base, TOK_PER_W)], probv)
        for v in range(TOK_PER_W // 16):
            banks = selv[pl.ds(v * 16, 16)]
            probs = probv[pl.ds(v * 16, 16)]
            flat = (lanes + (v * 16)) * NUM_BANKS + banks
            plsc.addupdate_scatter(ptile, [flat], probs)

    pltpu.sync_copy(ptile, p_hbm.at[pl.ds(base * NUM_BANKS,
                                          TOK_PER_W * NUM_BANKS)])


def _build_p(selT, probT):
    mesh = plsc.VectorSubcoreMesh(core_axis_name="c", subcore_axis_name="s")
    kfn = functools.partial(
        pl.kernel,
        mesh=mesh,
        out_type=jax.ShapeDtypeStruct((N * NUM_BANKS,), jnp.float32),
        scratch_types=[
            pltpu.VMEM((TOK_PER_W,), jnp.int32),
            pltpu.VMEM((TOK_PER_W,), jnp.float32),
            pltpu.VMEM((TOK_PER_W * NUM_BANKS,), jnp.float32),
        ],
        compiler_params=pltpu.CompilerParams(needs_layout_passes=False),
    )(_p_routing_sc)
    return kfn(selT, probT).reshape(N, NUM_BANKS)


def _mm_kernel(p_ref, x_ref, w_ref, b_ref, out_ref):
    p = p_ref[...]
    x = x_ref[...]
    pb = p.astype(jnp.bfloat16)
    acc = jnp.dot(pb, b_ref[...],
                  preferred_element_type=jnp.float32).astype(jnp.bfloat16)
    for b in range(NUM_BANKS):
        z = jnp.dot(x, w_ref[b], preferred_element_type=jnp.float32)
        acc = acc + pb[:, b:b + 1] * z.astype(jnp.bfloat16)
    out_ref[...] = acc.astype(jnp.float32)


def kernel(tensor, bank_selections, bank_probabilities, weights, bias):
    selT = bank_selections.astype(jnp.int32).T
    probT = bank_probabilities.T
    xb = tensor.astype(jnp.bfloat16)
    wb = weights.astype(jnp.bfloat16)
    bb = bias.astype(jnp.bfloat16)

    p_mat = _build_p(selT, probT)

    out = pl.pallas_call(
        _mm_kernel,
        grid=(NTILES,),
        in_specs=[
            pl.BlockSpec((TILE, NUM_BANKS), lambda t: (t, 0)),
            pl.BlockSpec((TILE, IN_FEATURES), lambda t: (t, 0)),
            pl.BlockSpec((NUM_BANKS, IN_FEATURES, OUT_FEATURES),
                         lambda t: (0, 0, 0)),
            pl.BlockSpec((NUM_BANKS, OUT_FEATURES), lambda t: (0, 0)),
        ],
        out_specs=pl.BlockSpec((TILE, OUT_FEATURES), lambda t: (t, 0)),
        out_shape=jax.ShapeDtypeStruct((N, OUT_FEATURES), jnp.float32),
        compiler_params=pltpu.CompilerParams(
            dimension_semantics=("parallel",),
        ),
    )(p_mat, xb, wb, bb)
    return out


# speedup vs baseline: 1.8945x; 1.8945x over previous
import jax
import jax.numpy as jnp
from jax.experimental import pallas as pl
from jax.experimental.pallas import tpu as pltpu

N = 2048
IN_FEATURES = 128
OUT_FEATURES = 128
NUM_BANKS = 64
TOP_K = 2
TILE = 256
NTILES = N // TILE


def _mm_kernel(sel_ref, prob_ref, x_ref, w_ref, b_ref, out_ref, wb_ref):
    t = pl.program_id(0)

    @pl.when(t == 0)
    def _():
        for b in range(NUM_BANKS):
            wb_ref[b] = w_ref[b].astype(jnp.bfloat16)

    sel = sel_ref[...]
    prob = prob_ref[...]
    banks = jax.lax.broadcasted_iota(jnp.int32, (TILE, NUM_BANKS), 1)
    p = jnp.zeros((TILE, NUM_BANKS), jnp.float32)
    for k in range(TOP_K):
        p += jnp.where(sel[:, k:k + 1] == banks, prob[:, k:k + 1], 0.0)

    x = x_ref[...].astype(jnp.bfloat16)
    pb = p.astype(jnp.bfloat16)
    bb = b_ref[...].astype(jnp.bfloat16)
    acc = jnp.dot(pb, bb,
                  preferred_element_type=jnp.float32).astype(jnp.bfloat16)
    for b in range(NUM_BANKS):
        z = jnp.dot(x, wb_ref[b], preferred_element_type=jnp.float32)
        acc = acc + pb[:, b:b + 1] * z.astype(jnp.bfloat16)
    out_ref[...] = acc.astype(jnp.float32)


def kernel(tensor, bank_selections, bank_probabilities, weights, bias):
    sel = bank_selections.astype(jnp.int32)

    out = pl.pallas_call(
        _mm_kernel,
        grid=(NTILES,),
        in_specs=[
            pl.BlockSpec((TILE, TOP_K), lambda t: (t, 0)),
            pl.BlockSpec((TILE, TOP_K), lambda t: (t, 0)),
            pl.BlockSpec((TILE, IN_FEATURES), lambda t: (t, 0)),
            pl.BlockSpec((NUM_BANKS, IN_FEATURES, OUT_FEATURES),
                         lambda t: (0, 0, 0)),
            pl.BlockSpec((NUM_BANKS, OUT_FEATURES), lambda t: (0, 0)),
        ],
        out_specs=pl.BlockSpec((TILE, OUT_FEATURES), lambda t: (t, 0)),
        out_shape=jax.ShapeDtypeStruct((N, OUT_FEATURES), jnp.float32),
        scratch_shapes=[
            pltpu.VMEM((NUM_BANKS, IN_FEATURES, OUT_FEATURES), jnp.bfloat16),
        ],
        compiler_params=pltpu.CompilerParams(
            dimension_semantics=("arbitrary",),
        ),
    )(sel, bank_probabilities, tensor, weights, bias)
    return out


# speedup vs baseline: 2.0447x; 1.0793x over previous
import jax
import jax.numpy as jnp
from jax.experimental import pallas as pl
from jax.experimental.pallas import tpu as pltpu

N = 2048
IN_FEATURES = 128
OUT_FEATURES = 128
NUM_BANKS = 64
TOP_K = 2
TILE = 512
NTILES = N // TILE


def _mm_kernel(sel_ref, prob_ref, x_ref, w_ref, b_ref, out_ref, wb_ref):
    t = pl.program_id(0)

    @pl.when(t == 0)
    def _():
        for b in range(NUM_BANKS):
            wb_ref[b] = w_ref[b].astype(jnp.bfloat16)

    sel = sel_ref[...]
    prob = prob_ref[...]
    banks = jax.lax.broadcasted_iota(jnp.int32, (TILE, NUM_BANKS), 1)
    p = jnp.zeros((TILE, NUM_BANKS), jnp.float32)
    for k in range(TOP_K):
        p += jnp.where(sel[:, k:k + 1] == banks, prob[:, k:k + 1], 0.0)

    x = x_ref[...].astype(jnp.bfloat16)
    pb = p.astype(jnp.bfloat16)
    bb = b_ref[...].astype(jnp.bfloat16)
    acc = jnp.dot(pb, bb,
                  preferred_element_type=jnp.float32).astype(jnp.bfloat16)
    for b in range(NUM_BANKS):
        z = jnp.dot(x, wb_ref[b], preferred_element_type=jnp.float32)
        acc = acc + pb[:, b:b + 1] * z.astype(jnp.bfloat16)
    out_ref[...] = acc.astype(jnp.float32)


def kernel(tensor, bank_selections, bank_probabilities, weights, bias):
    sel = bank_selections.astype(jnp.int32)

    out = pl.pallas_call(
        _mm_kernel,
        grid=(NTILES,),
        in_specs=[
            pl.BlockSpec((TILE, TOP_K), lambda t: (t, 0)),
            pl.BlockSpec((TILE, TOP_K), lambda t: (t, 0)),
            pl.BlockSpec((TILE, IN_FEATURES), lambda t: (t, 0)),
            pl.BlockSpec((NUM_BANKS, IN_FEATURES, OUT_FEATURES),
                         lambda t: (0, 0, 0)),
            pl.BlockSpec((NUM_BANKS, OUT_FEATURES), lambda t: (0, 0)),
        ],
        out_specs=pl.BlockSpec((TILE, OUT_FEATURES), lambda t: (t, 0)),
        out_shape=jax.ShapeDtypeStruct((N, OUT_FEATURES), jnp.float32),
        scratch_shapes=[
            pltpu.VMEM((NUM_BANKS, IN_FEATURES, OUT_FEATURES), jnp.bfloat16),
        ],
        compiler_params=pltpu.CompilerParams(
            dimension_semantics=("arbitrary",),
        ),
    )(sel, bank_probabilities, tensor, weights, bias)
    return out


# speedup vs baseline: 2.0635x; 1.0092x over previous
import jax
import jax.numpy as jnp
from jax.experimental import pallas as pl
from jax.experimental.pallas import tpu as pltpu

N = 2048
IN_FEATURES = 128
OUT_FEATURES = 128
NUM_BANKS = 64
TOP_K = 2
TILE = 1024
NTILES = N // TILE


def _mm_kernel(sel_ref, prob_ref, x_ref, w_ref, b_ref, out_ref, wb_ref):
    t = pl.program_id(0)

    @pl.when(t == 0)
    def _():
        for b in range(NUM_BANKS):
            wb_ref[b] = w_ref[b].astype(jnp.bfloat16)

    sel = sel_ref[...]
    prob = prob_ref[...]
    banks = jax.lax.broadcasted_iota(jnp.int32, (TILE, NUM_BANKS), 1)
    p = jnp.zeros((TILE, NUM_BANKS), jnp.float32)
    for k in range(TOP_K):
        p += jnp.where(sel[:, k:k + 1] == banks, prob[:, k:k + 1], 0.0)

    x = x_ref[...].astype(jnp.bfloat16)
    pb = p.astype(jnp.bfloat16)
    bb = b_ref[...].astype(jnp.bfloat16)
    acc = jnp.dot(pb, bb,
                  preferred_element_type=jnp.float32).astype(jnp.bfloat16)
    for b in range(NUM_BANKS):
        z = jnp.dot(x, wb_ref[b], preferred_element_type=jnp.float32)
        acc = acc + pb[:, b:b + 1] * z.astype(jnp.bfloat16)
    out_ref[...] = acc.astype(jnp.float32)


def kernel(tensor, bank_selections, bank_probabilities, weights, bias):
    sel = bank_selections.astype(jnp.int32)

    out = pl.pallas_call(
        _mm_kernel,
        grid=(NTILES,),
        in_specs=[
            pl.BlockSpec((TILE, TOP_K), lambda t: (t, 0)),
            pl.BlockSpec((TILE, TOP_K), lambda t: (t, 0)),
            pl.BlockSpec((TILE, IN_FEATURES), lambda t: (t, 0)),
            pl.BlockSpec((NUM_BANKS, IN_FEATURES, OUT_FEATURES),
                         lambda t: (0, 0, 0)),
            pl.BlockSpec((NUM_BANKS, OUT_FEATURES), lambda t: (0, 0)),
        ],
        out_specs=pl.BlockSpec((TILE, OUT_FEATURES), lambda t: (t, 0)),
        out_shape=jax.ShapeDtypeStruct((N, OUT_FEATURES), jnp.float32),
        scratch_shapes=[
            pltpu.VMEM((NUM_BANKS, IN_FEATURES, OUT_FEATURES), jnp.bfloat16),
        ],
        compiler_params=pltpu.CompilerParams(
            dimension_semantics=("arbitrary",),
        ),
    )(sel, bank_probabilities, tensor, weights, bias)
    return out
